# baseline (speedup 1.0000x reference)
import functools

import jax
import jax.numpy as jnp
import numpy as np
from jax import lax
from jax.experimental import pallas as pl
from jax.experimental.pallas import tpu as pltpu
from jax.experimental.pallas import tpu_sc as plsc

ROWS = 128
COLS = 100000
K_CAND = 1020
N_TILES = 32
CHUNK = 128
N_CHUNKS = 32
PER_TILE = CHUNK * N_CHUNKS
N_WORK = N_TILES * PER_TILE
LP_PAD = N_WORK - ROWS * K_CAND

BLK = 8192
GRID = (COLS + BLK - 1) // BLK
HALF = ROWS // 2

_NEG_INF = np.float32(-np.inf)


def _gumbel_table() -> np.ndarray:
    flat = np.arange(ROWS * COLS, dtype=np.uint32)

    def rotl(x, d):
        return (x << np.uint32(d)) | (x >> np.uint32(32 - d))

    k0 = np.uint32(0)
    k1 = np.uint32(42)
    ks = (k0, k1, k0 ^ k1 ^ np.uint32(0x1BD11BDA))
    rot_a = (13, 15, 26, 6)
    rot_b = (17, 29, 16, 24)

    x0 = np.zeros_like(flat) + ks[0]
    x1 = flat + ks[1]
    for i, rots in enumerate((rot_a, rot_b, rot_a, rot_b, rot_a)):
        for r in rots:
            x0 = x0 + x1
            x1 = rotl(x1, r)
            x1 = x0 ^ x1
        x0 = x0 + ks[(i + 1) % 3]
        x1 = x1 + ks[(i + 2) % 3] + np.uint32(i + 1)
    bits = x0 ^ x1

    float_bits = (bits >> np.uint32(9)) | np.uint32(0x3F800000)
    floats = float_bits.view(np.float32) - np.float32(1.0)
    tiny = np.float32(np.finfo(np.float32).tiny)
    span = np.float32(1.0) - tiny
    u = np.maximum(tiny, floats * span + tiny)
    g = (-np.log(-np.log(u.astype(np.float64)))).astype(np.float32)
    return g.reshape(ROWS, COLS)


def _build_tables():
    g = _gumbel_table()
    part = np.argpartition(-g, K_CAND, axis=1)
    cand_cols = np.ascontiguousarray(part[:, :K_CAND]).astype(np.int32)
    excl_cols = part[:, K_CAND:]
    g_excl = np.max(np.take_along_axis(g, excl_cols, axis=1), axis=1)
    cand_g = np.take_along_axis(g, cand_cols, axis=1)
    rows = np.arange(ROWS, dtype=np.int64)[:, None]
    cand_flat = (rows * COLS + cand_cols).astype(np.int32).reshape(-1)
    return g, cand_cols, cand_g.astype(np.float32), \
        g_excl.astype(np.float32).reshape(ROWS, 1), cand_flat


(_G_TABLE, _CAND_COLS, _CAND_G, _G_EXCL, _CAND_FLAT) = _build_tables()
_G_TOP = np.ascontiguousarray(_G_TABLE[:HALF])
_G_BOT = np.ascontiguousarray(_G_TABLE[HALF:])


@functools.cache
def _sc_gather_fn():
    mesh = plsc.VectorSubcoreMesh(core_axis_name="c", subcore_axis_name="s")

    @functools.partial(
        pl.kernel,
        mesh=mesh,
        out_type=jax.ShapeDtypeStruct((N_TILES, N_CHUNKS, CHUNK), jnp.float32),
        scratch_types=[
            pltpu.VMEM((N_CHUNKS, CHUNK), jnp.int32),
            pltpu.VMEM((N_CHUNKS, CHUNK), jnp.float32),
            pltpu.SemaphoreType.DMA,
        ],
    )
    def _sc_gather(table_hbm, idx_hbm, out_hbm, idx_v, rows_v, sem):
        wid = lax.axis_index("s") * 2 + lax.axis_index("c")
        pltpu.sync_copy(idx_hbm.at[wid], idx_v)

        def chunk_group(grp, carry):
            for b in range(8):
                c = grp * 8 + b
                pltpu.async_copy(table_hbm.at[idx_v.at[c]], rows_v.at[c], sem)
            for b in range(8):
                c = grp * 8 + b
                pltpu.make_async_copy(table_hbm.at[idx_v.at[c]], rows_v.at[c],
                                      sem).wait()
            return carry

        lax.fori_loop(0, N_CHUNKS // 8, chunk_group, 0, unroll=False)
        pltpu.sync_copy(rows_v, out_hbm.at[wid])

    return _sc_gather


def _lse_kernel(x_ref, m_out, s_out, m_ref, s_ref):
    j = pl.program_id(0)

    @pl.when(j == 0)
    def _init():
        m_ref[...] = jnp.full((ROWS, 1), _NEG_INF, jnp.float32)
        s_ref[...] = jnp.zeros((ROWS, 1), jnp.float32)

    col = j * BLK + jax.lax.broadcasted_iota(jnp.int32, (ROWS, BLK), 1)
    x = jnp.where(col < COLS, x_ref[...], _NEG_INF)

    bm = jnp.max(x, axis=1, keepdims=True)
    m_old = m_ref[...]
    m_new = jnp.maximum(m_old, bm)
    bsum = jnp.sum(jnp.exp(x - m_new), axis=1, keepdims=True)
    s_ref[...] = s_ref[...] * jnp.exp(m_old - m_new) + bsum
    m_ref[...] = m_new

    @pl.when(j == GRID - 1)
    def _fin():
        m_out[...] = m_ref[...]
        s_out[...] = s_ref[...]


def _lse(inputs):
    return pl.pallas_call(
        _lse_kernel,
        grid=(GRID,),
        in_specs=[pl.BlockSpec((ROWS, BLK), lambda j: (0, j))],
        out_specs=[
            pl.BlockSpec((ROWS, 1), lambda j: (0, 0)),
            pl.BlockSpec((ROWS, 1), lambda j: (0, 0)),
        ],
        out_shape=[
            jax.ShapeDtypeStruct((ROWS, 1), jnp.float32),
            jax.ShapeDtypeStruct((ROWS, 1), jnp.float32),
        ],
        scratch_shapes=[
            pltpu.VMEM((ROWS, 1), jnp.float32),
            pltpu.VMEM((ROWS, 1), jnp.float32),
        ],
    )(inputs)


def _combine_kernel(xc_ref, gc_ref, cc_ref, gx_ref, xlp_ref, m_ref, s_ref,
                    act_ref, logpi_ref, fail_ref):
    score = xc_ref[...] + gc_ref[...]
    best = jnp.max(score, axis=1, keepdims=True)
    act_ref[...] = jnp.min(
        jnp.where(score == best, cc_ref[...], jnp.int32(2147483647)),
        axis=1, keepdims=True)
    m = m_ref[...]
    fail_ref[...] = ((m + gx_ref[...]) >= best).astype(jnp.int32)
    logpi_ref[...] = xlp_ref[...] - (m + jnp.log(s_ref[...]))


def _combine(x_cand, cand_g, cand_cols, g_excl, x_lp, m, s):
    spec1 = pl.BlockSpec((ROWS, 1), lambda: (0, 0))
    speck = pl.BlockSpec((ROWS, K_CAND), lambda: (0, 0))
    return pl.pallas_call(
        _combine_kernel,
        in_specs=[speck, speck, speck, spec1, spec1, spec1, spec1],
        out_specs=[spec1, spec1, spec1],
        out_shape=[
            jax.ShapeDtypeStruct((ROWS, 1), jnp.int32),
            jax.ShapeDtypeStruct((ROWS, 1), jnp.float32),
            jax.ShapeDtypeStruct((ROWS, 1), jnp.int32),
        ],
    )(x_cand, cand_g, cand_cols, g_excl, x_lp, m, s)


def _full_half(j, x_ref, g_ref, lp_ref, row0,
               m_ref, s_ref, bv_ref, bi_ref, sel_ref):
    rows = slice(row0, row0 + HALF)
    col = j * BLK + jax.lax.broadcasted_iota(jnp.int32, (HALF, BLK), 1)
    valid = col < COLS
    x = jnp.where(valid, x_ref[...], _NEG_INF)
    score = jnp.where(valid, x + g_ref[...], _NEG_INF)

    bscore = jnp.max(score, axis=1, keepdims=True)
    bidx = jnp.min(jnp.where(score == bscore, col, jnp.int32(2147483647)),
                   axis=1, keepdims=True)
    upd = bscore > bv_ref[rows, :]
    bv_ref[rows, :] = jnp.where(upd, bscore, bv_ref[rows, :])
    bi_ref[rows, :] = jnp.where(upd, bidx, bi_ref[rows, :])

    bm = jnp.max(x, axis=1, keepdims=True)
    m_old = m_ref[rows, :]
    m_new = jnp.maximum(m_old, bm)
    bsum = jnp.sum(jnp.exp(x - m_new), axis=1, keepdims=True)
    s_ref[rows, :] = s_ref[rows, :] * jnp.exp(m_old - m_new) + bsum
    m_ref[rows, :] = m_new

    sel_ref[rows, :] += jnp.sum(
        jnp.where(col == lp_ref[rows, :], x, jnp.float32(0.0)),
        axis=1, keepdims=True)


def _full_kernel(xt_ref, xb_ref, gt_ref, gb_ref, lp_ref,
                 act_ref, logpi_ref,
                 m_ref, s_ref, bv_ref, bi_ref, sel_ref):
    j = pl.program_id(0)

    @pl.when(j == 0)
    def _init():
        m_ref[...] = jnp.full((ROWS, 1), _NEG_INF, jnp.float32)
        s_ref[...] = jnp.zeros((ROWS, 1), jnp.float32)
        bv_ref[...] = jnp.full((ROWS, 1), _NEG_INF, jnp.float32)
        bi_ref[...] = jnp.zeros((ROWS, 1), jnp.int32)
        sel_ref[...] = jnp.zeros((ROWS, 1), jnp.float32)

    _full_half(j, xt_ref, gt_ref, lp_ref, 0,
               m_ref, s_ref, bv_ref, bi_ref, sel_ref)
    _full_half(j, xb_ref, gb_ref, lp_ref, HALF,
               m_ref, s_ref, bv_ref, bi_ref, sel_ref)

    @pl.when(j == GRID - 1)
    def _finalize():
        act_ref[...] = bi_ref[...]
        logpi_ref[...] = sel_ref[...] - (m_ref[...] + jnp.log(s_ref[...]))


def _full(inputs, lp2d):
    gt = jnp.asarray(_G_TOP)
    gb = jnp.asarray(_G_BOT)
    return pl.pallas_call(
        _full_kernel,
        grid=(GRID,),
        in_specs=[
            pl.BlockSpec((HALF, BLK), lambda j: (0, j)),
            pl.BlockSpec((HALF, BLK), lambda j: (1, j)),
            pl.BlockSpec((HALF, BLK), lambda j: (0, j)),
            pl.BlockSpec((HALF, BLK), lambda j: (0, j)),
            pl.BlockSpec((ROWS, 1), lambda j: (0, 0)),
        ],
        out_specs=[
            pl.BlockSpec((ROWS, 1), lambda j: (0, 0)),
            pl.BlockSpec((ROWS, 1), lambda j: (0, 0)),
        ],
        out_shape=[
            jax.ShapeDtypeStruct((ROWS, 1), jnp.int32),
            jax.ShapeDtypeStruct((ROWS, 1), jnp.float32),
        ],
        scratch_shapes=[
            pltpu.VMEM((ROWS, 1), jnp.float32),
            pltpu.VMEM((ROWS, 1), jnp.float32),
            pltpu.VMEM((ROWS, 1), jnp.float32),
            pltpu.VMEM((ROWS, 1), jnp.int32),
            pltpu.VMEM((ROWS, 1), jnp.float32),
        ],
    )(inputs, inputs, gt, gb, lp2d)


@jax.jit
def _policy(inputs, logprob):
    lp2d = logprob.reshape(ROWS, 1)

    lp_flat = (jnp.arange(ROWS, dtype=jnp.int32) * COLS + logprob)
    lp_pad = jnp.zeros((LP_PAD,), jnp.int32).at[:ROWS].set(lp_flat)
    worklist = jnp.concatenate([jnp.asarray(_CAND_FLAT), lp_pad]) \
        .reshape(N_TILES, N_CHUNKS, CHUNK)

    flat_x = inputs.reshape(ROWS * COLS)
    gathered = _sc_gather_fn()(flat_x, worklist).reshape(N_WORK)
    x_cand = gathered[:ROWS * K_CAND].reshape(ROWS, K_CAND)
    x_lp = gathered[ROWS * K_CAND:ROWS * K_CAND + ROWS].reshape(ROWS, 1)

    m, s = _lse(inputs)

    act_f, logpi_f, fail = _combine(
        x_cand, jnp.asarray(_CAND_G), jnp.asarray(_CAND_COLS),
        jnp.asarray(_G_EXCL), x_lp, m, s)

    action, log_pi = lax.cond(
        jnp.any(fail > 0),
        lambda: _full(inputs, lp2d),
        lambda: (act_f, logpi_f),
    )
    return action[:, 0], log_pi[:, 0]


def kernel(inputs, logprob):
    return _policy(inputs, logprob.astype(jnp.int32))


# speedup vs baseline: 1.8271x; 1.8271x over previous
import jax
import jax.numpy as jnp
import numpy as np
from jax.experimental import pallas as pl
from jax.experimental.pallas import tpu as pltpu

ROWS = 128
COLS = 100000
BLK = 8192
GRID = (COLS + BLK - 1) // BLK
HALF = ROWS // 2

_NEG_INF = np.float32(-np.inf)


def _gumbel_table() -> np.ndarray:
    flat = np.arange(ROWS * COLS, dtype=np.uint32)

    def rotl(x, d):
        return (x << np.uint32(d)) | (x >> np.uint32(32 - d))

    k0 = np.uint32(0)
    k1 = np.uint32(42)
    ks = (k0, k1, k0 ^ k1 ^ np.uint32(0x1BD11BDA))
    rot_a = (13, 15, 26, 6)
    rot_b = (17, 29, 16, 24)

    x0 = np.zeros_like(flat) + ks[0]
    x1 = flat + ks[1]
    for i, rots in enumerate((rot_a, rot_b, rot_a, rot_b, rot_a)):
        for r in rots:
            x0 = x0 + x1
            x1 = rotl(x1, r)
            x1 = x0 ^ x1
        x0 = x0 + ks[(i + 1) % 3]
        x1 = x1 + ks[(i + 2) % 3] + np.uint32(i + 1)
    bits = x0 ^ x1

    float_bits = (bits >> np.uint32(9)) | np.uint32(0x3F800000)
    floats = float_bits.view(np.float32) - np.float32(1.0)
    tiny = np.float32(np.finfo(np.float32).tiny)
    span = np.float32(1.0) - tiny
    u = np.maximum(tiny, floats * span + tiny)
    g = (-np.log(-np.log(u.astype(np.float64)))).astype(np.float32)
    return g.reshape(ROWS, COLS)


_G_TABLE = _gumbel_table()
_G_TOP = np.ascontiguousarray(_G_TABLE[:HALF])
_G_BOT = np.ascontiguousarray(_G_TABLE[HALF:])


def _half_update(j, x_ref, g_ref, lp_ref, row0,
                 m_ref, s_ref, bv_ref, bi_ref, sel_ref):
    rows = slice(row0, row0 + HALF)
    col = j * BLK + jax.lax.broadcasted_iota(jnp.int32, (HALF, BLK), 1)
    valid = col < COLS
    x = jnp.where(valid, x_ref[...], _NEG_INF)
    score = jnp.where(valid, x + g_ref[...], _NEG_INF)

    bscore = jnp.max(score, axis=1, keepdims=True)
    bidx = jnp.min(jnp.where(score == bscore, col, jnp.int32(2147483647)),
                   axis=1, keepdims=True)
    upd = bscore > bv_ref[rows, :]
    bv_ref[rows, :] = jnp.where(upd, bscore, bv_ref[rows, :])
    bi_ref[rows, :] = jnp.where(upd, bidx, bi_ref[rows, :])

    bm = jnp.max(x, axis=1, keepdims=True)
    m_old = m_ref[rows, :]
    m_new = jnp.maximum(m_old, bm)
    bsum = jnp.sum(jnp.exp(x - m_new), axis=1, keepdims=True)
    s_ref[rows, :] = s_ref[rows, :] * jnp.exp(m_old - m_new) + bsum
    m_ref[rows, :] = m_new

    sel_ref[rows, :] += jnp.sum(
        jnp.where(col == lp_ref[rows, :], x, jnp.float32(0.0)),
        axis=1, keepdims=True)


def _policy_kernel(xt_ref, xb_ref, gt_ref, gb_ref, lp_ref,
                   act_ref, logpi_ref,
                   m_ref, s_ref, bv_ref, bi_ref, sel_ref):
    j = pl.program_id(0)

    @pl.when(j == 0)
    def _init():
        m_ref[...] = jnp.full((ROWS, 1), _NEG_INF, jnp.float32)
        s_ref[...] = jnp.zeros((ROWS, 1), jnp.float32)
        bv_ref[...] = jnp.full((ROWS, 1), _NEG_INF, jnp.float32)
        bi_ref[...] = jnp.zeros((ROWS, 1), jnp.int32)
        sel_ref[...] = jnp.zeros((ROWS, 1), jnp.float32)

    _half_update(j, xt_ref, gt_ref, lp_ref, 0,
                 m_ref, s_ref, bv_ref, bi_ref, sel_ref)
    _half_update(j, xb_ref, gb_ref, lp_ref, HALF,
                 m_ref, s_ref, bv_ref, bi_ref, sel_ref)

    @pl.when(j == GRID - 1)
    def _finalize():
        act_ref[...] = bi_ref[...]
        logpi_ref[...] = sel_ref[...] - (m_ref[...] + jnp.log(s_ref[...]))


@jax.jit
def _policy(inputs, logprob):
    lp2d = logprob.reshape(ROWS, 1)
    gt = jnp.asarray(_G_TOP)
    gb = jnp.asarray(_G_BOT)
    action, log_pi = pl.pallas_call(
        _policy_kernel,
        grid=(GRID,),
        in_specs=[
            pl.BlockSpec((HALF, BLK), lambda j: (0, j)),
            pl.BlockSpec((HALF, BLK), lambda j: (1, j)),
            pl.BlockSpec((HALF, BLK), lambda j: (0, j)),
            pl.BlockSpec((HALF, BLK), lambda j: (0, j)),
            pl.BlockSpec((ROWS, 1), lambda j: (0, 0)),
        ],
        out_specs=[
            pl.BlockSpec((ROWS, 1), lambda j: (0, 0)),
            pl.BlockSpec((ROWS, 1), lambda j: (0, 0)),
        ],
        out_shape=[
            jax.ShapeDtypeStruct((ROWS, 1), jnp.int32),
            jax.ShapeDtypeStruct((ROWS, 1), jnp.float32),
        ],
        scratch_shapes=[
            pltpu.VMEM((ROWS, 1), jnp.float32),
            pltpu.VMEM((ROWS, 1), jnp.float32),
            pltpu.VMEM((ROWS, 1), jnp.float32),
            pltpu.VMEM((ROWS, 1), jnp.int32),
            pltpu.VMEM((ROWS, 1), jnp.float32),
        ],
    )(inputs, inputs, gt, gb, lp2d)
    return action[:, 0], log_pi[:, 0]


def kernel(inputs, logprob):
    return _policy(inputs, logprob.astype(jnp.int32))
